# baseline (speedup 1.0000x reference)
import functools

import jax
import jax.numpy as jnp
from jax import lax
from jax.experimental import pallas as pl
from jax.experimental.pallas import tpu as pltpu
from jax.experimental.pallas import tpu_sc as plsc

NSEED = 10000
NH1 = 100000
D = 32
F = 128

GB_PAD = 102400
GNW = 32
GB_W = GB_PAD // GNW
GCHUNK = 640
GNCH = GB_W // GCHUNK
T4ROWS = 250000
RJ = 12800

RA = 2000
SA = RA // 10
RB = 2000


def _gather_body(t4_hbm, idx_hbm, out_hbm, idx_v, rows_v, sem):
    wid = lax.axis_index("s") * 2 + lax.axis_index("c")
    base = wid * GB_W
    pltpu.sync_copy(idx_hbm.at[pl.ds(base, GB_W)], idx_v)
    for g in range(GNCH):
        pltpu.async_copy(
            t4_hbm.at[idx_v.at[pl.ds(g * GCHUNK, GCHUNK)]],
            rows_v, sem,
        ).wait()
        pltpu.sync_copy(rows_v, out_hbm.at[pl.ds(base + g * GCHUNK, GCHUNK)])


def _sc_gather(t4, idx4_pad):
    mesh = plsc.VectorSubcoreMesh(core_axis_name="c", subcore_axis_name="s")
    return pl.kernel(
        _gather_body,
        mesh=mesh,
        out_type=jax.ShapeDtypeStruct((GB_PAD, F), jnp.float32),
        scratch_types=[
            pltpu.VMEM((GB_W,), jnp.int32),
            pltpu.VMEM((GCHUNK, F), jnp.float32),
            pltpu.SemaphoreType.DMA,
        ],
        cost_estimate=pl.CostEstimate(
            flops=0, bytes_accessed=110_000_000, transcendentals=0),
    )(t4, idx4_pad)


def _repack_body(tt_ref, o_ref, tmp_ref):
    eye = jnp.eye(D, dtype=jnp.float32)
    tmp_ref[...] = lax.dot_general(
        tt_ref[...], eye, (((0,), (0,)), ((), ())),
        preferred_element_type=jnp.float32)
    o_ref[...] = jnp.concatenate(
        [tmp_ref[pl.Slice(a, RJ // 4, 4), :] for a in range(4)], axis=1)


def _repack(table_t):
    return pl.pallas_call(
        _repack_body, grid=((1000000 + RJ - 1) // RJ,),
        in_specs=[pl.BlockSpec((D, RJ), lambda i: (0, i))],
        out_specs=pl.BlockSpec((RJ // 4, F), lambda i: (i, 0)),
        out_shape=jax.ShapeDtypeStruct((T4ROWS, F), jnp.float32),
        scratch_shapes=[pltpu.VMEM((RJ, D), jnp.float32)],
    )(table_t)


def _mean10_matrix():
    ii = lax.broadcasted_iota(jnp.int32, (SA, RA), 0)
    jj = lax.broadcasted_iota(jnp.int32, (SA, RA), 1)
    return jnp.where(jj // 10 == ii, jnp.float32(0.1), jnp.float32(0.0))


def _sum5(x2_ref):
    acc = x2_ref[pl.Slice(0, RA, 5), :]
    for k in range(1, 5):
        acc = acc + x2_ref[pl.Slice(k, RA, 5), :]
    return acc


def _a_body(x1_ref, x2_ref, emb4_ref, oh_ref, xb2_ref,
            al_ref, bl_ref, b5l_ref, bl0_ref,
            wb0lo_ref, wb0hi_ref, b5b_ref, bb0_ref,
            n1_ref, g1_ref, nb1_ref, gb1_ref):
    m10 = _mean10_matrix()
    x1 = x1_ref[...]
    c2 = jnp.dot(_sum5(x2_ref), b5l_ref[...],
                 preferred_element_type=jnp.float32)
    h1 = jnp.maximum(
        jnp.dot(x1, al_ref[...], preferred_element_type=jnp.float32)
        + c2 + bl0_ref[...], 0.0)
    g1_ref[...] = jnp.dot(m10, h1, preferred_element_type=jnp.float32)
    t = jnp.dot(x1, bl_ref[...], preferred_element_type=jnp.float32)
    n1_ref[...] = jnp.dot(m10, t, preferred_element_type=jnp.float32)
    emb4 = emb4_ref[...]
    oh = oh_ref[...]
    emb = (emb4[:, 0 * D:1 * D] * oh[:, 0:1]
           + emb4[:, 1 * D:2 * D] * oh[:, 1:2]
           + emb4[:, 2 * D:3 * D] * oh[:, 2:3]
           + emb4[:, 3 * D:4 * D] * oh[:, 3:4])
    c2b = jnp.dot(_sum5(xb2_ref), b5b_ref[...],
                  preferred_element_type=jnp.float32)
    hb1 = jnp.maximum(
        jnp.dot(emb, wb0lo_ref[...], preferred_element_type=jnp.float32)
        + c2b + bb0_ref[...], 0.0)
    gb1_ref[...] = jnp.dot(m10, hb1, preferred_element_type=jnp.float32)
    me = jnp.dot(m10, emb, preferred_element_type=jnp.float32)
    nb1_ref[...] = jnp.dot(me, wb0hi_ref[...], preferred_element_type=jnp.float32)

def _a1_body(x1_ref, x2_ref, al_ref, bl_ref, b5l_ref, bl0_ref, n1_ref, g1_ref):
    x1 = x1_ref[...]
    c2 = jnp.dot(_sum5(x2_ref), b5l_ref[...],
                 preferred_element_type=jnp.float32)
    h1 = jnp.maximum(
        jnp.dot(x1, al_ref[...], preferred_element_type=jnp.float32)
        + c2 + bl0_ref[...], 0.0)
    m10 = _mean10_matrix()
    g1_ref[...] = jnp.dot(m10, h1, preferred_element_type=jnp.float32)
    t = jnp.dot(x1, bl_ref[...], preferred_element_type=jnp.float32)
    n1_ref[...] = jnp.dot(m10, t, preferred_element_type=jnp.float32)


def _a2_body(emb4_ref, oh_ref, xb2_ref, wb0lo_ref, wb0hi_ref, b5b_ref, bb0_ref,
             nb1_ref, gb1_ref):
    emb4 = emb4_ref[...]
    oh = oh_ref[...]
    emb = (emb4[:, 0 * D:1 * D] * oh[:, 0:1]
           + emb4[:, 1 * D:2 * D] * oh[:, 1:2]
           + emb4[:, 2 * D:3 * D] * oh[:, 2:3]
           + emb4[:, 3 * D:4 * D] * oh[:, 3:4])
    c2 = jnp.dot(_sum5(xb2_ref), b5b_ref[...],
                 preferred_element_type=jnp.float32)
    hb1 = jnp.maximum(
        jnp.dot(emb, wb0lo_ref[...], preferred_element_type=jnp.float32)
        + c2 + bb0_ref[...], 0.0)
    m10 = _mean10_matrix()
    gb1_ref[...] = jnp.dot(m10, hb1, preferred_element_type=jnp.float32)
    me = jnp.dot(m10, emb, preferred_element_type=jnp.float32)
    nb1_ref[...] = jnp.dot(me, wb0hi_ref[...], preferred_element_type=jnp.float32)


def _b_body(x0_ref, xb0_ref, n1_ref, g1_ref, nb1_ref, gb1_ref,
            al_ref, ab_ref, wl1lo_ref, wl1hi_ref, wb1lo_ref, wb1hi_ref,
            bl0_ref, bb0_ref, bl1_ref, bb1_ref, w1f_ref, w1c_ref,
            w2_ref, b2_ref, cw_ref, out_ref):
    dot = functools.partial(jnp.dot, preferred_element_type=jnp.float32)
    h0 = jnp.maximum(dot(x0_ref[...], al_ref[...]) + n1_ref[...] + bl0_ref[...], 0.0)
    hb0 = jnp.maximum(dot(xb0_ref[...], ab_ref[...]) + nb1_ref[...] + bb0_ref[...], 0.0)
    o1 = dot(h0, wl1lo_ref[...]) + dot(g1_ref[...], wl1hi_ref[...]) + bl1_ref[...]
    ob1 = dot(hb0, wb1lo_ref[...]) + dot(gb1_ref[...], wb1hi_ref[...]) + bb1_ref[...]
    feats = [h0 * hb0, h0 * ob1, o1 * hb0, o1 * ob1, h0, o1]
    f6 = jnp.concatenate(feats, axis=1)
    lg6 = dot(f6, w1f_ref[...]) + w1c_ref[...]
    m = jnp.max(lg6, axis=1, keepdims=True)
    e6 = jnp.exp(lg6 - m)
    inv = 1.0 / jnp.sum(e6, axis=1, keepdims=True)
    out2 = (e6[:, 0:1] * feats[0] + e6[:, 1:2] * feats[1]
            + e6[:, 2:3] * feats[2] + e6[:, 3:4] * feats[3]
            + e6[:, 4:5] * feats[4] + e6[:, 5:6] * feats[5]) * inv
    cw = cw_ref[0:1, 0:1]
    out = cw * o1 + (1.0 - cw) * out2
    lg = dot(out, w2_ref[...]) + b2_ref[...]
    mm = jnp.max(lg, axis=1, keepdims=True)
    z = lg - mm
    lse = jnp.log(jnp.sum(jnp.exp(z), axis=1, keepdims=True))
    out_ref[...] = z - lse


def _full(shape):
    return pl.BlockSpec(shape, lambda i: (0, 0))


def _run_a(body, rows, hop2, weights):
    grid = (NH1 // RA,)
    d_rows = rows.shape[1]
    in_specs = [
        pl.BlockSpec((RA, d_rows), lambda i: (i, 0)),
        pl.BlockSpec((5 * RA, F), lambda i: (i, 0)),
    ] + [_full(w.shape) for w in weights]
    out_specs = [pl.BlockSpec((SA, D), lambda i: (i, 0))] * 2
    out_shape = [jax.ShapeDtypeStruct((NSEED, D), jnp.float32)] * 2
    return pl.pallas_call(
        body, grid=grid, in_specs=in_specs, out_specs=out_specs,
        out_shape=out_shape,
    )(rows, hop2, *weights)


def kernel(x_0, x_1, x_2, xb_0, xb_1, xb_2, weight_trans, table,
           Wl0, bl0, Wl1, bl1, Wb0, bb0, Wb1, bb1, W1, b1, W2, b2,
           com_weight=0.8):
    f32 = jnp.float32
    wt = weight_trans.astype(f32)
    al = wt @ Wl0[:D]
    bl = wt @ Wl0[D:]
    ab = wt @ Wb0[:D]
    bb = wt @ Wb0[D:]
    b5l = bl / 5.0
    b5b = bb / 5.0
    idx = xb_1.astype(jnp.int32)
    idx4_pad = jnp.concatenate(
        [idx // 4, jnp.zeros((GB_PAD - NH1,), jnp.int32)])
    onehot = jnp.equal(
        jnp.remainder(idx, 4)[:, None],
        jnp.arange(8, dtype=jnp.int32)[None, :]).astype(f32)

    t4 = _repack(table.astype(f32).T)
    emb4 = _sc_gather(t4, idx4_pad)

    n1, g1 = _run_a(_a1_body, x_1, x_2,
                    [al, bl, b5l, bl0.reshape(1, D)])
    emb4, n1, g1 = lax.optimization_barrier((emb4, n1, g1))

    wts_a2 = [Wb0[:D], Wb0[D:], b5b, bb0.reshape(1, D)]
    in_specs_a2 = [
        pl.BlockSpec((RA, F), lambda i: (i, 0)),
        pl.BlockSpec((RA, 8), lambda i: (i, 0)),
        pl.BlockSpec((5 * RA, F), lambda i: (i, 0)),
    ] + [_full(w.shape) for w in wts_a2]
    nb1, gb1 = pl.pallas_call(
        _a2_body, grid=(NH1 // RA,), in_specs=in_specs_a2,
        out_specs=[pl.BlockSpec((SA, D), lambda i: (i, 0))] * 2,
        out_shape=[jax.ShapeDtypeStruct((NSEED, D), jnp.float32)] * 2,
    )(emb4, onehot, xb_2, *wts_a2)

    w1f = jnp.zeros((6 * D, 8), f32)
    for i in range(6):
        w1f = w1f.at[i * D:(i + 1) * D, i].set(W1[:D, 0])
    w1c = jnp.full((1, 8), -1e30, f32).at[0, :6].set(W1[D:, 0] + b1[0])
    cw = jnp.asarray(com_weight, f32).reshape(1, 1)
    weights_b = [al, ab, Wl1[:D], Wl1[D:], Wb1[:D], Wb1[D:],
                 bl0.reshape(1, D), bb0.reshape(1, D),
                 bl1.reshape(1, D), bb1.reshape(1, D),
                 w1f, w1c, W2, b2.reshape(1, D), cw]

    grid = (NSEED // RB,)
    in_specs = (
        [pl.BlockSpec((RB, F), lambda i: (i, 0))] * 2
        + [pl.BlockSpec((RB, D), lambda i: (i, 0))] * 4
        + [_full(w.shape) for w in weights_b]
    )
    out = pl.pallas_call(
        _b_body, grid=grid, in_specs=in_specs,
        out_specs=pl.BlockSpec((RB, D), lambda i: (i, 0)),
        out_shape=jax.ShapeDtypeStruct((NSEED, D), jnp.float32),
    )(x_0, xb_0, n1, g1, nb1, gb1, *weights_b)
    return out


# speedup vs baseline: 1.0835x; 1.0835x over previous
import functools

import jax
import jax.numpy as jnp
from jax import lax
from jax.experimental import pallas as pl
from jax.experimental.pallas import tpu as pltpu
from jax.experimental.pallas import tpu_sc as plsc

NSEED = 10000
NH1 = 100000
D = 32
F = 128

GB_PAD = 102400
GNW = 32
GB_W = GB_PAD // GNW
GCHUNK = 320
GNCH = GB_W // GCHUNK
T4ROWS = 250000
RJ = 12800

RA = 2000
SA = RA // 10
RB = 2000


def _gather_body(t4_hbm, idx_hbm, out_hbm, idx_v, rows0, rows1, sem0, sem1):
    wid = lax.axis_index("s") * 2 + lax.axis_index("c")
    base = wid * GB_W
    pltpu.sync_copy(idx_hbm.at[pl.ds(base, GB_W)], idx_v)
    bufs = (rows0, rows1)
    sems = (sem0, sem1)
    cps = {0: pltpu.async_copy(
        t4_hbm.at[idx_v.at[pl.ds(0, GCHUNK)]], rows0, sem0)}
    for g in range(GNCH):
        if g + 1 < GNCH:
            cps[g + 1] = pltpu.async_copy(
                t4_hbm.at[idx_v.at[pl.ds((g + 1) * GCHUNK, GCHUNK)]],
                bufs[(g + 1) % 2], sems[(g + 1) % 2])
        cps[g].wait()
        pltpu.sync_copy(bufs[g % 2],
                        out_hbm.at[pl.ds(base + g * GCHUNK, GCHUNK)])


def _sc_gather(t4, idx4_pad):
    mesh = plsc.VectorSubcoreMesh(core_axis_name="c", subcore_axis_name="s")
    return pl.kernel(
        _gather_body,
        mesh=mesh,
        out_type=jax.ShapeDtypeStruct((GB_PAD, F), jnp.float32),
        scratch_types=[
            pltpu.VMEM((GB_W,), jnp.int32),
            pltpu.VMEM((GCHUNK, F), jnp.float32),
            pltpu.VMEM((GCHUNK, F), jnp.float32),
            pltpu.SemaphoreType.DMA,
            pltpu.SemaphoreType.DMA,
        ],
        cost_estimate=pl.CostEstimate(
            flops=0, bytes_accessed=110_000_000, transcendentals=0),
    )(t4, idx4_pad)


def _repack_body(tt_ref, o_ref, tmp_ref):
    eye = jnp.eye(D, dtype=jnp.float32)
    tmp_ref[...] = lax.dot_general(
        tt_ref[...], eye, (((0,), (0,)), ((), ())),
        preferred_element_type=jnp.float32)
    o_ref[...] = jnp.concatenate(
        [tmp_ref[pl.Slice(a, RJ // 4, 4), :] for a in range(4)], axis=1)


def _repack(table_t):
    return pl.pallas_call(
        _repack_body, grid=((1000000 + RJ - 1) // RJ,),
        in_specs=[pl.BlockSpec((D, RJ), lambda i: (0, i))],
        out_specs=pl.BlockSpec((RJ // 4, F), lambda i: (i, 0)),
        out_shape=jax.ShapeDtypeStruct((T4ROWS, F), jnp.float32),
        scratch_shapes=[pltpu.VMEM((RJ, D), jnp.float32)],
    )(table_t)


def _mean10_matrix():
    ii = lax.broadcasted_iota(jnp.int32, (SA, RA), 0)
    jj = lax.broadcasted_iota(jnp.int32, (SA, RA), 1)
    return jnp.where(jj // 10 == ii, jnp.float32(0.1), jnp.float32(0.0))


def _sum5(x2_ref):
    acc = x2_ref[pl.Slice(0, RA, 5), :]
    for k in range(1, 5):
        acc = acc + x2_ref[pl.Slice(k, RA, 5), :]
    return acc


def _a_body(x1_ref, x2_ref, emb4_ref, oh_ref, xb2_ref,
            al_ref, bl_ref, b5l_ref, bl0_ref,
            wb0lo_ref, wb0hi_ref, b5b_ref, bb0_ref,
            n1_ref, g1_ref, nb1_ref, gb1_ref):
    m10 = _mean10_matrix()
    x1 = x1_ref[...]
    c2 = jnp.dot(_sum5(x2_ref), b5l_ref[...],
                 preferred_element_type=jnp.float32)
    h1 = jnp.maximum(
        jnp.dot(x1, al_ref[...], preferred_element_type=jnp.float32)
        + c2 + bl0_ref[...], 0.0)
    g1_ref[...] = jnp.dot(m10, h1, preferred_element_type=jnp.float32)
    t = jnp.dot(x1, bl_ref[...], preferred_element_type=jnp.float32)
    n1_ref[...] = jnp.dot(m10, t, preferred_element_type=jnp.float32)
    emb4 = emb4_ref[...]
    oh = oh_ref[...]
    emb = (emb4[:, 0 * D:1 * D] * oh[:, 0:1]
           + emb4[:, 1 * D:2 * D] * oh[:, 1:2]
           + emb4[:, 2 * D:3 * D] * oh[:, 2:3]
           + emb4[:, 3 * D:4 * D] * oh[:, 3:4])
    c2b = jnp.dot(_sum5(xb2_ref), b5b_ref[...],
                  preferred_element_type=jnp.float32)
    hb1 = jnp.maximum(
        jnp.dot(emb, wb0lo_ref[...], preferred_element_type=jnp.float32)
        + c2b + bb0_ref[...], 0.0)
    gb1_ref[...] = jnp.dot(m10, hb1, preferred_element_type=jnp.float32)
    me = jnp.dot(m10, emb, preferred_element_type=jnp.float32)
    nb1_ref[...] = jnp.dot(me, wb0hi_ref[...], preferred_element_type=jnp.float32)

def _a1_body(x1_ref, x2_ref, al_ref, bl_ref, b5l_ref, bl0_ref, n1_ref, g1_ref):
    x1 = x1_ref[...]
    c2 = jnp.dot(_sum5(x2_ref), b5l_ref[...],
                 preferred_element_type=jnp.float32)
    h1 = jnp.maximum(
        jnp.dot(x1, al_ref[...], preferred_element_type=jnp.float32)
        + c2 + bl0_ref[...], 0.0)
    m10 = _mean10_matrix()
    g1_ref[...] = jnp.dot(m10, h1, preferred_element_type=jnp.float32)
    t = jnp.dot(x1, bl_ref[...], preferred_element_type=jnp.float32)
    n1_ref[...] = jnp.dot(m10, t, preferred_element_type=jnp.float32)


def _a2_body(emb4_ref, oh_ref, xb2_ref, wb0lo_ref, wb0hi_ref, b5b_ref, bb0_ref,
             nb1_ref, gb1_ref):
    emb4 = emb4_ref[...]
    oh = oh_ref[...]
    emb = (emb4[:, 0 * D:1 * D] * oh[:, 0:1]
           + emb4[:, 1 * D:2 * D] * oh[:, 1:2]
           + emb4[:, 2 * D:3 * D] * oh[:, 2:3]
           + emb4[:, 3 * D:4 * D] * oh[:, 3:4])
    c2 = jnp.dot(_sum5(xb2_ref), b5b_ref[...],
                 preferred_element_type=jnp.float32)
    hb1 = jnp.maximum(
        jnp.dot(emb, wb0lo_ref[...], preferred_element_type=jnp.float32)
        + c2 + bb0_ref[...], 0.0)
    m10 = _mean10_matrix()
    gb1_ref[...] = jnp.dot(m10, hb1, preferred_element_type=jnp.float32)
    me = jnp.dot(m10, emb, preferred_element_type=jnp.float32)
    nb1_ref[...] = jnp.dot(me, wb0hi_ref[...], preferred_element_type=jnp.float32)


def _b_body(x0_ref, xb0_ref, n1_ref, g1_ref, nb1_ref, gb1_ref,
            al_ref, ab_ref, wl1lo_ref, wl1hi_ref, wb1lo_ref, wb1hi_ref,
            bl0_ref, bb0_ref, bl1_ref, bb1_ref, w1f_ref, w1c_ref,
            w2_ref, b2_ref, cw_ref, out_ref):
    dot = functools.partial(jnp.dot, preferred_element_type=jnp.float32)
    h0 = jnp.maximum(dot(x0_ref[...], al_ref[...]) + n1_ref[...] + bl0_ref[...], 0.0)
    hb0 = jnp.maximum(dot(xb0_ref[...], ab_ref[...]) + nb1_ref[...] + bb0_ref[...], 0.0)
    o1 = dot(h0, wl1lo_ref[...]) + dot(g1_ref[...], wl1hi_ref[...]) + bl1_ref[...]
    ob1 = dot(hb0, wb1lo_ref[...]) + dot(gb1_ref[...], wb1hi_ref[...]) + bb1_ref[...]
    feats = [h0 * hb0, h0 * ob1, o1 * hb0, o1 * ob1, h0, o1]
    f6 = jnp.concatenate(feats, axis=1)
    lg6 = dot(f6, w1f_ref[...]) + w1c_ref[...]
    m = jnp.max(lg6, axis=1, keepdims=True)
    e6 = jnp.exp(lg6 - m)
    inv = 1.0 / jnp.sum(e6, axis=1, keepdims=True)
    out2 = (e6[:, 0:1] * feats[0] + e6[:, 1:2] * feats[1]
            + e6[:, 2:3] * feats[2] + e6[:, 3:4] * feats[3]
            + e6[:, 4:5] * feats[4] + e6[:, 5:6] * feats[5]) * inv
    cw = cw_ref[0:1, 0:1]
    out = cw * o1 + (1.0 - cw) * out2
    lg = dot(out, w2_ref[...]) + b2_ref[...]
    mm = jnp.max(lg, axis=1, keepdims=True)
    z = lg - mm
    lse = jnp.log(jnp.sum(jnp.exp(z), axis=1, keepdims=True))
    out_ref[...] = z - lse


def _full(shape):
    return pl.BlockSpec(shape, lambda i: (0, 0))


def _run_a(body, rows, hop2, weights):
    grid = (NH1 // RA,)
    d_rows = rows.shape[1]
    in_specs = [
        pl.BlockSpec((RA, d_rows), lambda i: (i, 0)),
        pl.BlockSpec((5 * RA, F), lambda i: (i, 0)),
    ] + [_full(w.shape) for w in weights]
    out_specs = [pl.BlockSpec((SA, D), lambda i: (i, 0))] * 2
    out_shape = [jax.ShapeDtypeStruct((NSEED, D), jnp.float32)] * 2
    return pl.pallas_call(
        body, grid=grid, in_specs=in_specs, out_specs=out_specs,
        out_shape=out_shape,
    )(rows, hop2, *weights)


def kernel(x_0, x_1, x_2, xb_0, xb_1, xb_2, weight_trans, table,
           Wl0, bl0, Wl1, bl1, Wb0, bb0, Wb1, bb1, W1, b1, W2, b2,
           com_weight=0.8):
    f32 = jnp.float32
    wt = weight_trans.astype(f32)
    al = wt @ Wl0[:D]
    bl = wt @ Wl0[D:]
    ab = wt @ Wb0[:D]
    bb = wt @ Wb0[D:]
    b5l = bl / 5.0
    b5b = bb / 5.0
    idx = xb_1.astype(jnp.int32)
    idx4_pad = jnp.concatenate(
        [idx // 4, jnp.zeros((GB_PAD - NH1,), jnp.int32)])
    onehot = jnp.equal(
        jnp.remainder(idx, 4)[:, None],
        jnp.arange(8, dtype=jnp.int32)[None, :]).astype(f32)

    t4 = _repack(table.astype(f32).T)
    emb4 = _sc_gather(t4, idx4_pad)

    wts_a = [al, bl, b5l, bl0.reshape(1, D),
             Wb0[:D], Wb0[D:], b5b, bb0.reshape(1, D)]
    in_specs_a = [
        pl.BlockSpec((RA, F), lambda i: (i, 0)),
        pl.BlockSpec((5 * RA, F), lambda i: (i, 0)),
        pl.BlockSpec((RA, F), lambda i: (i, 0)),
        pl.BlockSpec((RA, 8), lambda i: (i, 0)),
        pl.BlockSpec((5 * RA, F), lambda i: (i, 0)),
    ] + [_full(w.shape) for w in wts_a]
    n1, g1, nb1, gb1 = pl.pallas_call(
        _a_body, grid=(NH1 // RA,), in_specs=in_specs_a,
        out_specs=[pl.BlockSpec((SA, D), lambda i: (i, 0))] * 4,
        out_shape=[jax.ShapeDtypeStruct((NSEED, D), jnp.float32)] * 4,
    )(x_1, x_2, emb4, onehot, xb_2, *wts_a)

    w1f = jnp.zeros((6 * D, 8), f32)
    for i in range(6):
        w1f = w1f.at[i * D:(i + 1) * D, i].set(W1[:D, 0])
    w1c = jnp.full((1, 8), -1e30, f32).at[0, :6].set(W1[D:, 0] + b1[0])
    cw = jnp.asarray(com_weight, f32).reshape(1, 1)
    weights_b = [al, ab, Wl1[:D], Wl1[D:], Wb1[:D], Wb1[D:],
                 bl0.reshape(1, D), bb0.reshape(1, D),
                 bl1.reshape(1, D), bb1.reshape(1, D),
                 w1f, w1c, W2, b2.reshape(1, D), cw]

    grid = (NSEED // RB,)
    in_specs = (
        [pl.BlockSpec((RB, F), lambda i: (i, 0))] * 2
        + [pl.BlockSpec((RB, D), lambda i: (i, 0))] * 4
        + [_full(w.shape) for w in weights_b]
    )
    out = pl.pallas_call(
        _b_body, grid=grid, in_specs=in_specs,
        out_specs=pl.BlockSpec((RB, D), lambda i: (i, 0)),
        out_shape=jax.ShapeDtypeStruct((NSEED, D), jnp.float32),
    )(x_0, xb_0, n1, g1, nb1, gb1, *weights_b)
    return out
